# baseline (speedup 1.0000x reference)
import functools

import jax
import jax.numpy as jnp
from jax import lax
from jax.experimental import pallas as pl
from jax.experimental.pallas import tpu as pltpu
from jax.experimental.pallas import tpu_sc as plsc

N = 10000
E = 160000
D = 128
NSUB = 16
CHUNK = 80
EPS = E // NSUB
NCH = EPS // CHUNK
RPS = N // NSUB
HW = 16
BLK = 1000
GRID = N // BLK


NBUF = 5
NGRP = NCH // NBUF


def _hist_body(idx4, ones_h, zdeg, deg8, idxb, ones_v, acc, isem, ssem):
    core = lax.axis_index("c")
    sub = lax.axis_index("s")
    row0 = sub * RPS
    for j in range(4):
        pltpu.async_copy(idx4.at[core * 4 + j, sub], idxb.at[j], isem.at[j])
    pltpu.sync_copy(ones_h, ones_v)
    for j in range(4):
        pltpu.make_async_copy(idx4.at[core * 4 + j, sub], idxb.at[j],
                              isem.at[j]).wait()
    for j in range(4):
        pltpu.sync_copy(zdeg, acc.at[pl.ds(row0, RPS)])
        plsc.subcore_barrier()

        def s_start(c, b):
            pltpu.async_copy(ones_v, acc.at[idxb.at[j, c]],
                             ssem.at[b], add=True)

        def s_wait(c, b):
            pltpu.make_async_copy(ones_v, acc.at[idxb.at[j, c]],
                                  ssem.at[b]).wait()

        for b in range(NBUF):
            s_start(b, b)

        @pl.loop(1, NGRP)
        def _(q):
            base = q * NBUF
            for b in range(NBUF):
                s_wait(base + b - NBUF, b)
                s_start(base + b, b)

        for b in range(NBUF):
            s_wait((NGRP - 1) * NBUF + b, b)

        plsc.subcore_barrier()
        g = core * 4 + j
        pltpu.sync_copy(acc.at[pl.ds(row0, RPS)],
                        deg8.at[g, pl.ds(row0, RPS)])
        plsc.subcore_barrier()


DH = D // 2


def _agg_body(h24, idx4, zrows, out4, sidxb, didxb, rows, acc, isem, gsem, ssem):
    core = lax.axis_index("c")
    sub = lax.axis_index("s")
    row0 = sub * RPS
    for r in range(2):
        rel = core * 2 + r
        pltpu.async_copy(idx4.at[2 * rel, sub], sidxb, isem.at[0])
        pltpu.async_copy(idx4.at[2 * rel + 1, sub], didxb, isem.at[1])
        pltpu.make_async_copy(idx4.at[2 * rel, sub], sidxb, isem.at[0]).wait()
        pltpu.make_async_copy(idx4.at[2 * rel + 1, sub], didxb, isem.at[1]).wait()
        for hh in range(2):
            src = h24.at[hh, rel]
            dst = out4.at[rel, pl.ds(sub * RPS, RPS), pl.ds(hh * DH, DH)]
            pltpu.sync_copy(zrows, acc.at[pl.ds(row0, RPS)])
            plsc.subcore_barrier()

            def g_start(c, b):
                pltpu.async_copy(src.at[sidxb.at[c]], rows.at[b], gsem.at[b])

            def g_wait(c, b):
                pltpu.make_async_copy(src.at[sidxb.at[c]], rows.at[b],
                                      gsem.at[b]).wait()

            def s_start(c, b):
                pltpu.async_copy(rows.at[b], acc.at[didxb.at[c]], ssem.at[b],
                                 add=True)

            def s_wait(c, b):
                pltpu.make_async_copy(rows.at[b], acc.at[didxb.at[c]],
                                      ssem.at[b]).wait()

            for c in range(NBUF):
                g_start(c, c)
                if c >= 2:
                    g_wait(c - 2, c - 2)
                    s_start(c - 2, c - 2)

            @pl.loop(1, NGRP)
            def _(q):
                base = q * NBUF
                for b in range(NBUF):
                    c = base + b
                    s_wait(c - NBUF, b)
                    g_start(c, b)
                    bs = (b - 2) % NBUF
                    g_wait(c - 2, bs)
                    s_start(c - 2, bs)

            last = NCH - NBUF
            for c in (NCH - 2, NCH - 1):
                g_wait(c, c % NBUF)
                s_start(c, c % NBUF)
            for b in range(NBUF):
                s_wait(last + b, b)

            plsc.subcore_barrier()
            pltpu.sync_copy(acc.at[pl.ds(row0, RPS)], dst)
            plsc.subcore_barrier()


@functools.lru_cache(maxsize=None)
def _sc_kernels():
    mesh = plsc.VectorSubcoreMesh(core_axis_name="c", subcore_axis_name="s",
                                  num_cores=2, num_subcores=NSUB)
    cp = pltpu.CompilerParams(use_tc_tiling_on_sc=False)
    hist = pl.kernel(
        _hist_body,
        out_type=jax.ShapeDtypeStruct((8, N, HW), jnp.float32),
        mesh=mesh,
        scratch_types=[
            pltpu.VMEM((4, NCH, CHUNK), jnp.int32),
            pltpu.VMEM((CHUNK, HW), jnp.float32),
            pltpu.VMEM_SHARED((N, HW), jnp.float32),
            pltpu.SemaphoreType.DMA((4,)),
            pltpu.SemaphoreType.DMA((NBUF,)),
        ],
        compiler_params=cp,
    )
    agg = pl.kernel(
        _agg_body,
        out_type=jax.ShapeDtypeStruct((4, N, D), jnp.float32),
        mesh=mesh,
        scratch_types=[
            pltpu.VMEM((NCH, CHUNK), jnp.int32),
            pltpu.VMEM((NCH, CHUNK), jnp.int32),
            pltpu.VMEM((NBUF, CHUNK, DH), jnp.float32),
            pltpu.VMEM_SHARED((N, DH), jnp.float32),
            pltpu.SemaphoreType.DMA((2,)),
            pltpu.SemaphoreType.DMA((NBUF,)),
            pltpu.SemaphoreType.DMA((NBUF,)),
        ],
        compiler_params=cp,
    )
    return hist, agg


def _scale(deg_blk):
    return lax.rsqrt(jnp.maximum(deg_blk[:, 0:1], 1.0))


def _prep_body(xw_ref, xd_ref, deg_ref, h24_ref):
    xw = xw_ref[0]
    xd = xd_ref[0]
    for r, x in ((0, xw), (1, xw), (2, xw), (3, xd)):
        h24_ref[0, r] = x * _scale(deg_ref[2 * r])


def _layer_mm(agg_ref, deg_ref, ww_ref, wd_ref, bw_ref, bd_ref, nout):
    accw = jnp.broadcast_to(bw_ref[...], (BLK, nout))
    for k, (r, g) in enumerate(((0, 1), (1, 3), (3, 7))):
        h = agg_ref[r] * _scale(deg_ref[g])
        accw = accw + jnp.dot(h, ww_ref[k], preferred_element_type=jnp.float32)
    hd = agg_ref[2] * _scale(deg_ref[5])
    accd = jnp.dot(hd, wd_ref[0], preferred_element_type=jnp.float32) + bd_ref[...]
    return jnp.maximum(accw, 0.0), jnp.maximum(accd, 0.0)


def _finish1_body(agg_ref, deg_ref, ww_ref, wd_ref, bw_ref, bd_ref, h24_ref):
    xw, xd = _layer_mm(agg_ref, deg_ref, ww_ref.at[0], wd_ref.at[0],
                       bw_ref.at[0], bd_ref.at[0], DH)
    for r, x in ((0, xw), (1, xw), (2, xw), (3, xd)):
        h24_ref[0, r] = x * _scale(deg_ref[2 * r])


def _finish2_body(agg_ref, deg_ref, ww_ref, wd_ref, bw_ref, bd_ref,
                  wlin_ref, blin_ref, ow_ref, od_ref):
    xw, xd = _layer_mm(agg_ref, deg_ref, ww_ref, wd_ref, bw_ref, bd_ref, D)
    wlin = wlin_ref[...]
    blin = blin_ref[...]
    ow_ref[...] = jax.nn.sigmoid(
        jnp.dot(xw, wlin, preferred_element_type=jnp.float32) + blin)
    od_ref[...] = jax.nn.sigmoid(
        jnp.dot(xd, wlin, preferred_element_type=jnp.float32) + blin)


_X_SPEC = pl.BlockSpec((BLK, D), lambda i: (i, 0))
_DEG_SPEC = pl.BlockSpec((8, BLK, HW), lambda i: (0, i, 0))
_H4_SPEC = pl.BlockSpec((4, BLK, D), lambda i: (0, i, 0))
_H24_SPEC = pl.BlockSpec((2, 4, BLK, DH), lambda i: (0, 0, i, 0))
_B_SPEC = pl.BlockSpec((1, D), lambda i: (0, 0))
_F32 = jnp.float32

_XH_SPEC = pl.BlockSpec((1, BLK, DH), lambda i, h: (h, i, 0))
_DEG2_SPEC = pl.BlockSpec((8, BLK, HW), lambda i, h: (0, i, 0))
_H24_OUT_SPEC = pl.BlockSpec((1, 4, BLK, DH), lambda i, h: (h, 0, i, 0))

_prep = pl.pallas_call(
    _prep_body,
    grid=(GRID, 2),
    in_specs=[_XH_SPEC, _XH_SPEC, _DEG2_SPEC],
    out_specs=_H24_OUT_SPEC,
    out_shape=jax.ShapeDtypeStruct((2, 4, N, DH), _F32),
)

_finish1 = pl.pallas_call(
    _finish1_body,
    grid=(GRID, 2),
    in_specs=[pl.BlockSpec((4, BLK, D), lambda i, h: (0, i, 0)),
              _DEG2_SPEC,
              pl.BlockSpec((1, 3, D, DH), lambda i, h: (h, 0, 0, 0)),
              pl.BlockSpec((1, 1, D, DH), lambda i, h: (h, 0, 0, 0)),
              pl.BlockSpec((1, 1, DH), lambda i, h: (h, 0, 0)),
              pl.BlockSpec((1, 1, DH), lambda i, h: (h, 0, 0))],
    out_specs=_H24_OUT_SPEC,
    out_shape=jax.ShapeDtypeStruct((2, 4, N, DH), _F32),
)

_finish2 = pl.pallas_call(
    _finish2_body,
    grid=(GRID,),
    in_specs=[_H4_SPEC, _DEG_SPEC,
              pl.BlockSpec((3, D, D), lambda i: (0, 0, 0)),
              pl.BlockSpec((1, D, D), lambda i: (0, 0, 0)),
              _B_SPEC, _B_SPEC,
              pl.BlockSpec((D, 1), lambda i: (0, 0)),
              pl.BlockSpec((1, 1), lambda i: (0, 0))],
    out_specs=[pl.BlockSpec((BLK, 1), lambda i: (i, 0)),
               pl.BlockSpec((BLK, 1), lambda i: (i, 0))],
    out_shape=[jax.ShapeDtypeStruct((N, 1), _F32),
               jax.ShapeDtypeStruct((N, 1), _F32)],
)


def kernel(x_word, x_doc, edge_ww, edge_wwr, edge_wd, edge_dw,
           W1_ww, b1_ww, W1_wwr, b1_wwr, W1_wd, b1_wd, W1_dw, b1_dw,
           W2_ww, b2_ww, W2_wwr, b2_wwr, W2_wd, b2_wd, W2_dw, b2_dw,
           W_lin, b_lin):
    idx8 = jnp.concatenate([edge_ww, edge_wwr, edge_wd, edge_dw], axis=0)
    idx4 = idx8.reshape(8, NSUB, NCH, CHUNK)
    ones_h = jnp.ones((CHUNK, HW), _F32)
    zdeg = jnp.zeros((RPS, HW), _F32)
    zrows = jnp.zeros((RPS, DH), _F32)

    w1w = jnp.stack([W1_ww, W1_wwr, W1_dw])
    w1d = W1_wd[None]
    b1w = (b1_ww + b1_wwr + b1_dw).reshape(1, D)
    b1d = b1_wd.reshape(1, D)
    w2w = jnp.stack([W2_ww, W2_wwr, W2_dw])
    w2d = W2_wd[None]
    b2w = (b2_ww + b2_wwr + b2_dw).reshape(1, D)
    b2d = b2_wd.reshape(1, D)
    blin = b_lin.reshape(1, 1)

    _hist, _agg = _sc_kernels()
    xw2 = x_word.reshape(N, 2, DH).transpose(1, 0, 2)
    xd2 = x_doc.reshape(N, 2, DH).transpose(1, 0, 2)
    deg8 = _hist(idx4, ones_h, zdeg)
    h1 = _prep(xw2, xd2, deg8)
    agg1 = _agg(h1, idx4, zrows)
    w1w2 = w1w.reshape(3, D, 2, DH).transpose(2, 0, 1, 3)
    w1d2 = w1d.reshape(1, D, 2, DH).transpose(2, 0, 1, 3)
    b1w2 = b1w.reshape(1, 2, DH).transpose(1, 0, 2)
    b1d2 = b1d.reshape(1, 2, DH).transpose(1, 0, 2)
    h2 = _finish1(agg1, deg8, w1w2, w1d2, b1w2, b1d2)
    agg2 = _agg(h2, idx4, zrows)
    out_word, out_doc = _finish2(agg2, deg8, w2w, w2d, b2w, b2d, W_lin, blin)
    return (out_word, out_doc)


# speedup vs baseline: 1.1211x; 1.1211x over previous
import functools

import jax
import jax.numpy as jnp
from jax import lax
from jax.experimental import pallas as pl
from jax.experimental.pallas import tpu as pltpu
from jax.experimental.pallas import tpu_sc as plsc

N = 10000
E = 160000
D = 128
NSUB = 16
CHUNK = 80
EPS = E // NSUB
NCH = EPS // CHUNK
RPS = N // NSUB
HW = 16
BLK = 1000
GRID = N // BLK


NBUF = 5
NGRP = NCH // NBUF


def _hist_body(idx4, ones_h, zdeg, deg8, idxb, ones_v, acc, isem, ssem):
    core = lax.axis_index("c")
    sub = lax.axis_index("s")
    row0 = sub * RPS
    for j in range(4):
        pltpu.async_copy(idx4.at[core * 4 + j, sub], idxb.at[j], isem.at[j])
    pltpu.sync_copy(ones_h, ones_v)
    for j in range(4):
        pltpu.make_async_copy(idx4.at[core * 4 + j, sub], idxb.at[j],
                              isem.at[j]).wait()
    for j in range(4):
        pltpu.sync_copy(zdeg, acc.at[pl.ds(row0, RPS)])
        plsc.subcore_barrier()

        def s_start(c, b):
            pltpu.async_copy(ones_v, acc.at[idxb.at[j, c]],
                             ssem.at[b], add=True)

        def s_wait(c, b):
            pltpu.make_async_copy(ones_v, acc.at[idxb.at[j, c]],
                                  ssem.at[b]).wait()

        for b in range(NBUF):
            s_start(b, b)

        @pl.loop(1, NGRP)
        def _(q):
            base = q * NBUF
            for b in range(NBUF):
                s_wait(base + b - NBUF, b)
                s_start(base + b, b)

        for b in range(NBUF):
            s_wait((NGRP - 1) * NBUF + b, b)

        plsc.subcore_barrier()
        g = core * 4 + j
        pltpu.sync_copy(acc.at[pl.ds(row0, RPS)],
                        deg8.at[g, pl.ds(row0, RPS)])
        plsc.subcore_barrier()


DH = D // 2


def _agg_body(h24, idx4, zrows, out4, sidxb, didxb, rows, acc, isem, gsem, ssem):
    core = lax.axis_index("c")
    sub = lax.axis_index("s")
    row0 = sub * RPS
    for r in range(2):
        rel = core * 2 + r
        pltpu.async_copy(idx4.at[2 * rel, sub], sidxb, isem.at[0])
        pltpu.async_copy(idx4.at[2 * rel + 1, sub], didxb, isem.at[1])
        pltpu.make_async_copy(idx4.at[2 * rel, sub], sidxb, isem.at[0]).wait()
        pltpu.make_async_copy(idx4.at[2 * rel + 1, sub], didxb, isem.at[1]).wait()
        for hh in range(2):
            src = h24.at[hh, rel]
            dst = out4.at[rel, pl.ds(sub * RPS, RPS), pl.ds(hh * DH, DH)]
            pltpu.sync_copy(zrows, acc.at[pl.ds(row0, RPS)])
            plsc.subcore_barrier()

            def g_start(c, b):
                pltpu.async_copy(src.at[sidxb.at[c]], rows.at[b], gsem.at[b])

            def g_wait(c, b):
                pltpu.make_async_copy(src.at[sidxb.at[c]], rows.at[b],
                                      gsem.at[b]).wait()

            def s_start(c, b):
                pltpu.async_copy(rows.at[b], acc.at[didxb.at[c]], ssem.at[b],
                                 add=True)

            def s_wait(c, b):
                pltpu.make_async_copy(rows.at[b], acc.at[didxb.at[c]],
                                      ssem.at[b]).wait()

            for c in range(NBUF):
                g_start(c, c)
                if c >= 3:
                    g_wait(c - 3, c - 3)
                    s_start(c - 3, c - 3)

            @pl.loop(1, NGRP)
            def _(q):
                base = q * NBUF
                for b in range(NBUF):
                    c = base + b
                    s_wait(c - NBUF, b)
                    g_start(c, b)
                    bs = (b - 3) % NBUF
                    g_wait(c - 3, bs)
                    s_start(c - 3, bs)

            last = NCH - NBUF
            for c in (NCH - 3, NCH - 2, NCH - 1):
                g_wait(c, c % NBUF)
                s_start(c, c % NBUF)
            for b in range(NBUF):
                s_wait(last + b, b)

            plsc.subcore_barrier()
            pltpu.sync_copy(acc.at[pl.ds(row0, RPS)], dst)
            plsc.subcore_barrier()


@functools.lru_cache(maxsize=None)
def _sc_kernels():
    mesh = plsc.VectorSubcoreMesh(core_axis_name="c", subcore_axis_name="s",
                                  num_cores=2, num_subcores=NSUB)
    cp = pltpu.CompilerParams(use_tc_tiling_on_sc=False)
    hist = pl.kernel(
        _hist_body,
        out_type=jax.ShapeDtypeStruct((8, N, HW), jnp.float32),
        mesh=mesh,
        scratch_types=[
            pltpu.VMEM((4, NCH, CHUNK), jnp.int32),
            pltpu.VMEM((CHUNK, HW), jnp.float32),
            pltpu.VMEM_SHARED((N, HW), jnp.float32),
            pltpu.SemaphoreType.DMA((4,)),
            pltpu.SemaphoreType.DMA((NBUF,)),
        ],
        compiler_params=cp,
    )
    agg = pl.kernel(
        _agg_body,
        out_type=jax.ShapeDtypeStruct((4, N, D), jnp.float32),
        mesh=mesh,
        scratch_types=[
            pltpu.VMEM((NCH, CHUNK), jnp.int32),
            pltpu.VMEM((NCH, CHUNK), jnp.int32),
            pltpu.VMEM((NBUF, CHUNK, DH), jnp.float32),
            pltpu.VMEM_SHARED((N, DH), jnp.float32),
            pltpu.SemaphoreType.DMA((2,)),
            pltpu.SemaphoreType.DMA((NBUF,)),
            pltpu.SemaphoreType.DMA((NBUF,)),
        ],
        compiler_params=cp,
    )
    return hist, agg


def _scale(deg_blk):
    return lax.rsqrt(jnp.maximum(deg_blk[:, 0:1], 1.0))


def _prep_body(xw_ref, xd_ref, deg_ref, h24_ref):
    xw = xw_ref[...]
    xd = xd_ref[...]
    for r, x in ((0, xw), (1, xw), (2, xw), (3, xd)):
        h = x * _scale(deg_ref[2 * r])
        h24_ref[0, r] = h[:, :DH]
        h24_ref[1, r] = h[:, DH:]


def _layer_mm(agg_ref, deg_ref, ww_ref, wd_ref, bw_ref, bd_ref):
    accw = jnp.broadcast_to(bw_ref[...], (BLK, D))
    for k, (r, g) in enumerate(((0, 1), (1, 3), (3, 7))):
        h = agg_ref[r] * _scale(deg_ref[g])
        accw = accw + jnp.dot(h, ww_ref[k], preferred_element_type=jnp.float32)
    hd = agg_ref[2] * _scale(deg_ref[5])
    accd = jnp.dot(hd, wd_ref[0], preferred_element_type=jnp.float32) + bd_ref[...]
    return jnp.maximum(accw, 0.0), jnp.maximum(accd, 0.0)


def _finish1_body(agg_ref, deg_ref, ww_ref, wd_ref, bw_ref, bd_ref, h24_ref):
    xw, xd = _layer_mm(agg_ref, deg_ref, ww_ref, wd_ref, bw_ref, bd_ref)
    for r, x in ((0, xw), (1, xw), (2, xw), (3, xd)):
        h = x * _scale(deg_ref[2 * r])
        h24_ref[0, r] = h[:, :DH]
        h24_ref[1, r] = h[:, DH:]


def _finish2_body(agg_ref, deg_ref, ww_ref, wd_ref, bw_ref, bd_ref,
                  wlin_ref, blin_ref, ow_ref, od_ref):
    xw, xd = _layer_mm(agg_ref, deg_ref, ww_ref, wd_ref, bw_ref, bd_ref)
    wlin = wlin_ref[...]
    blin = blin_ref[...]
    ow_ref[...] = jax.nn.sigmoid(
        jnp.dot(xw, wlin, preferred_element_type=jnp.float32) + blin)
    od_ref[...] = jax.nn.sigmoid(
        jnp.dot(xd, wlin, preferred_element_type=jnp.float32) + blin)


_X_SPEC = pl.BlockSpec((BLK, D), lambda i: (i, 0))
_DEG_SPEC = pl.BlockSpec((8, BLK, HW), lambda i: (0, i, 0))
_H4_SPEC = pl.BlockSpec((4, BLK, D), lambda i: (0, i, 0))
_H24_SPEC = pl.BlockSpec((2, 4, BLK, DH), lambda i: (0, 0, i, 0))
_B_SPEC = pl.BlockSpec((1, D), lambda i: (0, 0))
_F32 = jnp.float32

_H24_SPEC = pl.BlockSpec((2, 4, BLK, DH), lambda i: (0, 0, i, 0))

_prep = pl.pallas_call(
    _prep_body,
    grid=(GRID,),
    in_specs=[_X_SPEC, _X_SPEC, _DEG_SPEC],
    out_specs=_H24_SPEC,
    out_shape=jax.ShapeDtypeStruct((2, 4, N, DH), _F32),
)

_finish1 = pl.pallas_call(
    _finish1_body,
    grid=(GRID,),
    in_specs=[_H4_SPEC, _DEG_SPEC,
              pl.BlockSpec((3, D, D), lambda i: (0, 0, 0)),
              pl.BlockSpec((1, D, D), lambda i: (0, 0, 0)),
              _B_SPEC, _B_SPEC],
    out_specs=_H24_SPEC,
    out_shape=jax.ShapeDtypeStruct((2, 4, N, DH), _F32),
)

_finish2 = pl.pallas_call(
    _finish2_body,
    grid=(GRID,),
    in_specs=[_H4_SPEC, _DEG_SPEC,
              pl.BlockSpec((3, D, D), lambda i: (0, 0, 0)),
              pl.BlockSpec((1, D, D), lambda i: (0, 0, 0)),
              _B_SPEC, _B_SPEC,
              pl.BlockSpec((D, 1), lambda i: (0, 0)),
              pl.BlockSpec((1, 1), lambda i: (0, 0))],
    out_specs=[pl.BlockSpec((BLK, 1), lambda i: (i, 0)),
               pl.BlockSpec((BLK, 1), lambda i: (i, 0))],
    out_shape=[jax.ShapeDtypeStruct((N, 1), _F32),
               jax.ShapeDtypeStruct((N, 1), _F32)],
)


def kernel(x_word, x_doc, edge_ww, edge_wwr, edge_wd, edge_dw,
           W1_ww, b1_ww, W1_wwr, b1_wwr, W1_wd, b1_wd, W1_dw, b1_dw,
           W2_ww, b2_ww, W2_wwr, b2_wwr, W2_wd, b2_wd, W2_dw, b2_dw,
           W_lin, b_lin):
    idx8 = jnp.concatenate([edge_ww, edge_wwr, edge_wd, edge_dw], axis=0)
    idx4 = idx8.reshape(8, NSUB, NCH, CHUNK)
    ones_h = jnp.ones((CHUNK, HW), _F32)
    zdeg = jnp.zeros((RPS, HW), _F32)
    zrows = jnp.zeros((RPS, DH), _F32)

    w1w = jnp.stack([W1_ww, W1_wwr, W1_dw])
    w1d = W1_wd[None]
    b1w = (b1_ww + b1_wwr + b1_dw).reshape(1, D)
    b1d = b1_wd.reshape(1, D)
    w2w = jnp.stack([W2_ww, W2_wwr, W2_dw])
    w2d = W2_wd[None]
    b2w = (b2_ww + b2_wwr + b2_dw).reshape(1, D)
    b2d = b2_wd.reshape(1, D)
    blin = b_lin.reshape(1, 1)

    _hist, _agg = _sc_kernels()
    deg8 = _hist(idx4, ones_h, zdeg)
    h1 = _prep(x_word, x_doc, deg8)
    agg1 = _agg(h1, idx4, zrows)
    h2 = _finish1(agg1, deg8, w1w, w1d, b1w, b1d)
    agg2 = _agg(h2, idx4, zrows)
    out_word, out_doc = _finish2(agg2, deg8, w2w, w2d, b2w, b2d, W_lin, blin)
    return (out_word, out_doc)


# speedup vs baseline: 1.1450x; 1.0214x over previous
import functools

import jax
import jax.numpy as jnp
from jax import lax
from jax.experimental import pallas as pl
from jax.experimental.pallas import tpu as pltpu
from jax.experimental.pallas import tpu_sc as plsc

N = 10000
E = 160000
D = 128
NSUB = 16
CHUNK = 80
EPS = E // NSUB
NCH = EPS // CHUNK
RPS = N // NSUB
HW = 16
BLK = 1000
GRID = N // BLK


NBUF = 5
NGRP = NCH // NBUF


def _hist_body(idx4, ones_h, zdeg, deg8, idxb, ones_v, acc, isem, ssem):
    core = lax.axis_index("c")
    sub = lax.axis_index("s")
    row0 = sub * RPS
    for j in range(4):
        pltpu.async_copy(idx4.at[core * 4 + j, sub], idxb.at[j], isem.at[j])
    pltpu.sync_copy(ones_h, ones_v)
    for j in range(4):
        pltpu.make_async_copy(idx4.at[core * 4 + j, sub], idxb.at[j],
                              isem.at[j]).wait()
    for j in range(4):
        pltpu.sync_copy(zdeg, acc.at[pl.ds(row0, RPS)])
        plsc.subcore_barrier()

        def s_start(c, b):
            pltpu.async_copy(ones_v, acc.at[idxb.at[j, c]],
                             ssem.at[b], add=True)

        def s_wait(c, b):
            pltpu.make_async_copy(ones_v, acc.at[idxb.at[j, c]],
                                  ssem.at[b]).wait()

        for b in range(NBUF):
            s_start(b, b)

        @pl.loop(1, NGRP)
        def _(q):
            base = q * NBUF
            for b in range(NBUF):
                s_wait(base + b - NBUF, b)
                s_start(base + b, b)

        for b in range(NBUF):
            s_wait((NGRP - 1) * NBUF + b, b)

        plsc.subcore_barrier()
        g = core * 4 + j
        pltpu.sync_copy(acc.at[pl.ds(row0, RPS)],
                        deg8.at[g, pl.ds(row0, RPS)])
        plsc.subcore_barrier()


DH = D // 2


def _agg_body(h24, idx4, zrows, out4, sidxb, didxb, rows, acc, isem, gsem, ssem):
    core = lax.axis_index("c")
    sub = lax.axis_index("s")
    row0 = sub * RPS
    for r in range(2):
        rel = core * 2 + r
        pltpu.async_copy(idx4.at[2 * rel, sub], sidxb, isem.at[0])
        pltpu.async_copy(idx4.at[2 * rel + 1, sub], didxb, isem.at[1])
        pltpu.make_async_copy(idx4.at[2 * rel, sub], sidxb, isem.at[0]).wait()
        pltpu.make_async_copy(idx4.at[2 * rel + 1, sub], didxb, isem.at[1]).wait()
        for hh in range(2):
            src = h24.at[hh, rel]
            dst = out4.at[rel, pl.ds(sub * RPS, RPS), pl.ds(hh * DH, DH)]
            pltpu.sync_copy(zrows, acc.at[pl.ds(row0, RPS)])
            plsc.subcore_barrier()

            def g_start(c, b):
                pltpu.async_copy(src.at[sidxb.at[c]], rows.at[b], gsem.at[b])

            def g_wait(c, b):
                pltpu.make_async_copy(src.at[sidxb.at[c]], rows.at[b],
                                      gsem.at[b]).wait()

            def s_start(c, b):
                pltpu.async_copy(rows.at[b], acc.at[didxb.at[c]], ssem.at[b],
                                 add=True)

            def s_wait(c, b):
                pltpu.make_async_copy(rows.at[b], acc.at[didxb.at[c]],
                                      ssem.at[b]).wait()

            for c in range(NBUF):
                g_start(c, c)
                if c >= 4:
                    g_wait(c - 4, c - 4)
                    s_start(c - 4, c - 4)

            @pl.loop(1, NGRP)
            def _(q):
                base = q * NBUF
                for b in range(NBUF):
                    c = base + b
                    s_wait(c - NBUF, b)
                    g_start(c, b)
                    bs = (b - 4) % NBUF
                    g_wait(c - 4, bs)
                    s_start(c - 4, bs)

            last = NCH - NBUF
            for c in (NCH - 4, NCH - 3, NCH - 2, NCH - 1):
                g_wait(c, c % NBUF)
                s_start(c, c % NBUF)
            for b in range(NBUF):
                s_wait(last + b, b)

            plsc.subcore_barrier()
            pltpu.sync_copy(acc.at[pl.ds(row0, RPS)], dst)
            plsc.subcore_barrier()


@functools.lru_cache(maxsize=None)
def _sc_kernels():
    mesh = plsc.VectorSubcoreMesh(core_axis_name="c", subcore_axis_name="s",
                                  num_cores=2, num_subcores=NSUB)
    cp = pltpu.CompilerParams(use_tc_tiling_on_sc=False)
    hist = pl.kernel(
        _hist_body,
        out_type=jax.ShapeDtypeStruct((8, N, HW), jnp.float32),
        mesh=mesh,
        scratch_types=[
            pltpu.VMEM((4, NCH, CHUNK), jnp.int32),
            pltpu.VMEM((CHUNK, HW), jnp.float32),
            pltpu.VMEM_SHARED((N, HW), jnp.float32),
            pltpu.SemaphoreType.DMA((4,)),
            pltpu.SemaphoreType.DMA((NBUF,)),
        ],
        compiler_params=cp,
    )
    agg = pl.kernel(
        _agg_body,
        out_type=jax.ShapeDtypeStruct((4, N, D), jnp.float32),
        mesh=mesh,
        scratch_types=[
            pltpu.VMEM((NCH, CHUNK), jnp.int32),
            pltpu.VMEM((NCH, CHUNK), jnp.int32),
            pltpu.VMEM((NBUF, CHUNK, DH), jnp.float32),
            pltpu.VMEM_SHARED((N, DH), jnp.float32),
            pltpu.SemaphoreType.DMA((2,)),
            pltpu.SemaphoreType.DMA((NBUF,)),
            pltpu.SemaphoreType.DMA((NBUF,)),
        ],
        compiler_params=cp,
    )
    return hist, agg


def _scale(deg_blk):
    return lax.rsqrt(jnp.maximum(deg_blk[:, 0:1], 1.0))


def _prep_body(xw_ref, xd_ref, deg_ref, h24_ref):
    xw = xw_ref[...]
    xd = xd_ref[...]
    for r, x in ((0, xw), (1, xw), (2, xw), (3, xd)):
        h = x * _scale(deg_ref[2 * r])
        h24_ref[0, r] = h[:, :DH]
        h24_ref[1, r] = h[:, DH:]


def _layer_mm(agg_ref, deg_ref, ww_ref, wd_ref, bw_ref, bd_ref):
    accw = jnp.broadcast_to(bw_ref[...], (BLK, D))
    for k, (r, g) in enumerate(((0, 1), (1, 3), (3, 7))):
        h = agg_ref[r] * _scale(deg_ref[g])
        accw = accw + jnp.dot(h, ww_ref[k], preferred_element_type=jnp.float32)
    hd = agg_ref[2] * _scale(deg_ref[5])
    accd = jnp.dot(hd, wd_ref[0], preferred_element_type=jnp.float32) + bd_ref[...]
    return jnp.maximum(accw, 0.0), jnp.maximum(accd, 0.0)


def _finish1_body(agg_ref, deg_ref, ww_ref, wd_ref, bw_ref, bd_ref, h24_ref):
    xw, xd = _layer_mm(agg_ref, deg_ref, ww_ref, wd_ref, bw_ref, bd_ref)
    for r, x in ((0, xw), (1, xw), (2, xw), (3, xd)):
        h = x * _scale(deg_ref[2 * r])
        h24_ref[0, r] = h[:, :DH]
        h24_ref[1, r] = h[:, DH:]


def _finish2_body(agg_ref, deg_ref, ww_ref, wd_ref, bw_ref, bd_ref,
                  wlin_ref, blin_ref, ow_ref, od_ref):
    xw, xd = _layer_mm(agg_ref, deg_ref, ww_ref, wd_ref, bw_ref, bd_ref)
    wlin = wlin_ref[...]
    blin = blin_ref[...]
    ow_ref[...] = jax.nn.sigmoid(
        jnp.dot(xw, wlin, preferred_element_type=jnp.float32) + blin)
    od_ref[...] = jax.nn.sigmoid(
        jnp.dot(xd, wlin, preferred_element_type=jnp.float32) + blin)


_X_SPEC = pl.BlockSpec((BLK, D), lambda i: (i, 0))
_DEG_SPEC = pl.BlockSpec((8, BLK, HW), lambda i: (0, i, 0))
_H4_SPEC = pl.BlockSpec((4, BLK, D), lambda i: (0, i, 0))
_H24_SPEC = pl.BlockSpec((2, 4, BLK, DH), lambda i: (0, 0, i, 0))
_B_SPEC = pl.BlockSpec((1, D), lambda i: (0, 0))
_F32 = jnp.float32

_H24_SPEC = pl.BlockSpec((2, 4, BLK, DH), lambda i: (0, 0, i, 0))

_prep = pl.pallas_call(
    _prep_body,
    grid=(GRID,),
    in_specs=[_X_SPEC, _X_SPEC, _DEG_SPEC],
    out_specs=_H24_SPEC,
    out_shape=jax.ShapeDtypeStruct((2, 4, N, DH), _F32),
)

_finish1 = pl.pallas_call(
    _finish1_body,
    grid=(GRID,),
    in_specs=[_H4_SPEC, _DEG_SPEC,
              pl.BlockSpec((3, D, D), lambda i: (0, 0, 0)),
              pl.BlockSpec((1, D, D), lambda i: (0, 0, 0)),
              _B_SPEC, _B_SPEC],
    out_specs=_H24_SPEC,
    out_shape=jax.ShapeDtypeStruct((2, 4, N, DH), _F32),
)

_finish2 = pl.pallas_call(
    _finish2_body,
    grid=(GRID,),
    in_specs=[_H4_SPEC, _DEG_SPEC,
              pl.BlockSpec((3, D, D), lambda i: (0, 0, 0)),
              pl.BlockSpec((1, D, D), lambda i: (0, 0, 0)),
              _B_SPEC, _B_SPEC,
              pl.BlockSpec((D, 1), lambda i: (0, 0)),
              pl.BlockSpec((1, 1), lambda i: (0, 0))],
    out_specs=[pl.BlockSpec((BLK, 1), lambda i: (i, 0)),
               pl.BlockSpec((BLK, 1), lambda i: (i, 0))],
    out_shape=[jax.ShapeDtypeStruct((N, 1), _F32),
               jax.ShapeDtypeStruct((N, 1), _F32)],
)


def kernel(x_word, x_doc, edge_ww, edge_wwr, edge_wd, edge_dw,
           W1_ww, b1_ww, W1_wwr, b1_wwr, W1_wd, b1_wd, W1_dw, b1_dw,
           W2_ww, b2_ww, W2_wwr, b2_wwr, W2_wd, b2_wd, W2_dw, b2_dw,
           W_lin, b_lin):
    idx8 = jnp.concatenate([edge_ww, edge_wwr, edge_wd, edge_dw], axis=0)
    idx4 = idx8.reshape(8, NSUB, NCH, CHUNK)
    ones_h = jnp.ones((CHUNK, HW), _F32)
    zdeg = jnp.zeros((RPS, HW), _F32)
    zrows = jnp.zeros((RPS, DH), _F32)

    w1w = jnp.stack([W1_ww, W1_wwr, W1_dw])
    w1d = W1_wd[None]
    b1w = (b1_ww + b1_wwr + b1_dw).reshape(1, D)
    b1d = b1_wd.reshape(1, D)
    w2w = jnp.stack([W2_ww, W2_wwr, W2_dw])
    w2d = W2_wd[None]
    b2w = (b2_ww + b2_wwr + b2_dw).reshape(1, D)
    b2d = b2_wd.reshape(1, D)
    blin = b_lin.reshape(1, 1)

    _hist, _agg = _sc_kernels()
    deg8 = _hist(idx4, ones_h, zdeg)
    h1 = _prep(x_word, x_doc, deg8)
    agg1 = _agg(h1, idx4, zrows)
    h2 = _finish1(agg1, deg8, w1w, w1d, b1w, b1d)
    agg2 = _agg(h2, idx4, zrows)
    out_word, out_doc = _finish2(agg2, deg8, w2w, w2d, b2w, b2d, W_lin, blin)
    return (out_word, out_doc)
